# baseline (speedup 1.0000x reference)
import functools

import jax
import jax.numpy as jnp
from jax import lax
from jax.experimental import pallas as pl
from jax.experimental.pallas import tpu as pltpu
from jax.experimental.pallas import tpu_sc as plsc

N = 7
R = 16
E = 640000
HID = 16
OUT = 8
RS = R * N
BINS = N * RS
L = 16
NW = 16
EPW = E // NW
CH = 8000
NCH = EPW // CH
CVECS = CH // L


def _sc_hist_kernel(src_hbm, dst_hbm, typ_hbm, out_hbm,
                    s0, d0, t0, s1, d1, t1, hist_v, final_v,
                    sem0, sem1):
    wid = lax.axis_index("s")
    base = wid * EPW
    bufs = ((s0, d0, t0), (s1, d1, t1))
    sems = (sem0, sem1)

    def start(c):
        sb, db, tb = bufs[c % 2]
        off = base + c * CH
        sem = sems[c % 2]
        return (pltpu.async_copy(src_hbm.at[pl.ds(off, CH)], sb, sem),
                pltpu.async_copy(dst_hbm.at[pl.ds(off, CH)], db, sem),
                pltpu.async_copy(typ_hbm.at[pl.ds(off, CH)], tb, sem))

    cps = start(0)

    zeros = jnp.zeros((L,), jnp.float32)

    @plsc.parallel_loop(0, BINS, unroll=8)
    def _(i):
        hist_v[pl.ds(i * L, L)] = zeros

    lane_off = lax.broadcasted_iota(jnp.int32, (L,), 0) * BINS
    ones = jnp.ones((L,), jnp.float32)

    def process(sb, db, tb):
        @plsc.parallel_loop(0, CVECS, unroll=8)
        def _(i):
            o = i * L
            s = sb[pl.ds(o, L)]
            d = db[pl.ds(o, L)]
            t = tb[pl.ds(o, L)]
            plsc.addupdate_scatter(
                hist_v, [(d * RS + t * N) + (s + lane_off)], ones)

    for c in range(NCH):
        for cp in cps:
            cp.wait()
        if c + 1 < NCH:
            cps = start(c + 1)
        process(*bufs[c % 2])

    for i in range(BINS // L):
        o = i * L
        acc = hist_v[pl.ds(o, L)]
        for l in range(1, L):
            acc = acc + hist_v[pl.ds(l * BINS + o, L)]
        final_v[i // (RS // L), pl.ds((i % (RS // L)) * L, L)] = acc

    pltpu.sync_copy(final_v, out_hbm.at[wid])


def _make_sc_hist():
    return pl.kernel(
        _sc_hist_kernel,
        mesh=plsc.VectorSubcoreMesh(core_axis_name="c", subcore_axis_name="s",
                                    num_cores=1),
        out_type=jax.ShapeDtypeStruct((NW, N, RS), jnp.float32),
        compiler_params=pltpu.CompilerParams(needs_layout_passes=False),
        scratch_types=[
            pltpu.VMEM((CH,), jnp.int32),
            pltpu.VMEM((CH,), jnp.int32),
            pltpu.VMEM((CH,), jnp.int32),
            pltpu.VMEM((CH,), jnp.int32),
            pltpu.VMEM((CH,), jnp.int32),
            pltpu.VMEM((CH,), jnp.int32),
            pltpu.VMEM((L * BINS,), jnp.float32),
            pltpu.VMEM((N, RS), jnp.float32),
            pltpu.SemaphoreType.DMA,
            pltpu.SemaphoreType.DMA,
        ],
    )


def _tc_finish_body(parts_ref, w1_ref, r1_ref, b1_ref, w2_ref, rt2_ref,
                    b2_ref, out_ref):
    counts = jnp.sum(parts_ref[...], axis=0)
    g = (lax.broadcasted_iota(jnp.int32, (RS, R), 0) // N
         == lax.broadcasted_iota(jnp.int32, (RS, R), 1)).astype(jnp.float32)
    gt = (lax.broadcasted_iota(jnp.int32, (R, RS), 1) // N
          == lax.broadcasted_iota(jnp.int32, (R, RS), 0)).astype(jnp.float32)
    cnt = jnp.dot(counts, g, preferred_element_type=jnp.float32)
    denom = jnp.maximum(
        jnp.dot(cnt, gt, preferred_element_type=jnp.float32), 1.0)
    q = counts / denom
    agg1 = jnp.dot(q, w1_ref[...], preferred_element_type=jnp.float32)
    h = jnp.maximum(agg1 + r1_ref[...] + b1_ref[...], 0.0)
    w2h = jnp.concatenate(
        [jnp.dot(h, w2_ref[r], preferred_element_type=jnp.float32)
         for r in range(R)], axis=0)
    acc = (jnp.dot(q, w2h, preferred_element_type=jnp.float32)
           + jnp.dot(h, rt2_ref[...], preferred_element_type=jnp.float32)
           + b2_ref[...])
    m = jnp.max(acc, axis=1, keepdims=True)
    e = jnp.exp(acc - m)
    lse = jnp.log(jnp.sum(e, axis=1, keepdims=True))
    out_ref[...] = acc - m - lse


def kernel(x, edge_index, edge_type, weight1, root1, bias1, weight2, root2,
           bias2):
    del x
    parts = _make_sc_hist()(edge_index[0], edge_index[1], edge_type)
    return pl.pallas_call(
        _tc_finish_body,
        out_shape=jax.ShapeDtypeStruct((N, OUT), jnp.float32),
    )(parts, weight1.reshape(RS, HID), root1, bias1.reshape(1, HID),
      weight2, root2, bias2.reshape(1, OUT))


# speedup vs baseline: 1.1340x; 1.1340x over previous
import functools

import jax
import jax.numpy as jnp
from jax import lax
from jax.experimental import pallas as pl
from jax.experimental.pallas import tpu as pltpu
from jax.experimental.pallas import tpu_sc as plsc

N = 7
R = 16
E = 640000
HID = 16
OUT = 8
RS = R * N
BINS = N * RS
L = 16
NW = 16
TILE = 512
NTILES = E // TILE
TPW = NTILES // NW
TCH = 13
CW = TCH * TILE
NCH = TPW // TCH


def _sc_hist_kernel(ei_hbm, typ_hbm, out_hbm,
                    e0, t0, e1, t1, hist_v, final_v,
                    sem0, sem1):
    wid = lax.axis_index("s")
    base = TPW * TILE * wid
    ebufs = (e0, e1)
    tbufs = (t0, t1)
    sems = (sem0, sem1)

    def start(c):
        off = base + c * CW
        sem = sems[c % 2]
        return (pltpu.async_copy(ei_hbm.at[:, pl.ds(off, CW)],
                                 ebufs[c % 2], sem),
                pltpu.async_copy(typ_hbm.at[pl.ds(off, CW)],
                                 tbufs[c % 2], sem))

    cps = start(0)

    zeros = jnp.zeros((L,), jnp.float32)

    @plsc.parallel_loop(0, BINS, unroll=8)
    def _(i):
        hist_v[pl.ds(i * L, L)] = zeros

    lane_off = lax.broadcasted_iota(jnp.int32, (L,), 0) * BINS
    ones = jnp.ones((L,), jnp.float32)

    def process(eb, tb, nvec):
        @plsc.parallel_loop(0, nvec, unroll=8)
        def _(i):
            o = i * L
            s = eb[0, pl.ds(o, L)]
            d = eb[1, pl.ds(o, L)]
            t = tb[pl.ds(o, L)]
            plsc.addupdate_scatter(
                hist_v, [(d * RS + t * N) + (s + lane_off)], ones)

    for c in range(NCH):
        for cp in cps:
            cp.wait()
        if c + 1 < NCH:
            cps = start(c + 1)
        process(ebufs[c % 2], tbufs[c % 2], CW // L)

    @pl.when(wid < 2)
    def _():
        off = (NW * TPW + wid) * TILE
        cpe = pltpu.async_copy(ei_hbm.at[:, pl.ds(off, TILE)],
                               e0.at[:, pl.ds(0, TILE)], sem0)
        cpt = pltpu.async_copy(typ_hbm.at[pl.ds(off, TILE)],
                               t0.at[pl.ds(0, TILE)], sem0)
        cpe.wait()
        cpt.wait()
        process(e0, t0, TILE // L)

    for i in range(BINS // L):
        o = i * L
        acc = hist_v[pl.ds(o, L)]
        for l in range(1, L):
            acc = acc + hist_v[pl.ds(l * BINS + o, L)]
        final_v[i // (RS // L), pl.ds((i % (RS // L)) * L, L)] = acc

    pltpu.sync_copy(final_v, out_hbm.at[wid])


def _make_sc_hist():
    return pl.kernel(
        _sc_hist_kernel,
        mesh=plsc.VectorSubcoreMesh(core_axis_name="c", subcore_axis_name="s",
                                    num_cores=1),
        out_type=jax.ShapeDtypeStruct((NW, N, RS), jnp.float32),
        compiler_params=pltpu.CompilerParams(needs_layout_passes=False),
        scratch_types=[
            pltpu.VMEM((2, CW), jnp.int32),
            pltpu.VMEM((CW,), jnp.int32),
            pltpu.VMEM((2, CW), jnp.int32),
            pltpu.VMEM((CW,), jnp.int32),
            pltpu.VMEM((L * BINS,), jnp.float32),
            pltpu.VMEM((N, RS), jnp.float32),
            pltpu.SemaphoreType.DMA,
            pltpu.SemaphoreType.DMA,
        ],
    )


def _tc_finish_body(parts_ref, w1_ref, r1_ref, b1_ref, w2_ref, rt2_ref,
                    b2_ref, out_ref):
    counts = jnp.sum(parts_ref[...], axis=0)
    g = (lax.broadcasted_iota(jnp.int32, (RS, R), 0) // N
         == lax.broadcasted_iota(jnp.int32, (RS, R), 1)).astype(jnp.float32)
    gt = (lax.broadcasted_iota(jnp.int32, (R, RS), 1) // N
          == lax.broadcasted_iota(jnp.int32, (R, RS), 0)).astype(jnp.float32)
    cnt = jnp.dot(counts, g, preferred_element_type=jnp.float32)
    denom = jnp.maximum(
        jnp.dot(cnt, gt, preferred_element_type=jnp.float32), 1.0)
    q = counts / denom
    agg1 = jnp.dot(q, w1_ref[...], preferred_element_type=jnp.float32)
    h = jnp.maximum(agg1 + r1_ref[...] + b1_ref[...], 0.0)
    w2h = jnp.concatenate(
        [jnp.dot(h, w2_ref[r], preferred_element_type=jnp.float32)
         for r in range(R)], axis=0)
    acc = (jnp.dot(q, w2h, preferred_element_type=jnp.float32)
           + jnp.dot(h, rt2_ref[...], preferred_element_type=jnp.float32)
           + b2_ref[...])
    m = jnp.max(acc, axis=1, keepdims=True)
    e = jnp.exp(acc - m)
    lse = jnp.log(jnp.sum(e, axis=1, keepdims=True))
    out_ref[...] = acc - m - lse


def kernel(x, edge_index, edge_type, weight1, root1, bias1, weight2, root2,
           bias2):
    del x
    parts = _make_sc_hist()(edge_index, edge_type)
    return pl.pallas_call(
        _tc_finish_body,
        out_shape=jax.ShapeDtypeStruct((N, OUT), jnp.float32),
    )(parts, weight1.reshape(RS, HID), root1, bias1.reshape(1, HID),
      weight2, root2, bias2.reshape(1, OUT))


# speedup vs baseline: 1.1949x; 1.0537x over previous
import functools

import jax
import jax.numpy as jnp
from jax import lax
from jax.experimental import pallas as pl
from jax.experimental.pallas import tpu as pltpu
from jax.experimental.pallas import tpu_sc as plsc

N = 7
R = 16
E = 640000
HID = 16
OUT = 8
RS = R * N
BP = 128
BROWS = 8
BINSP = BROWS * BP
L = 16
NW = 16
TILE = 512
NTILES = E // TILE
TPW = NTILES // NW
TCH = 13
CW = TCH * TILE
NCH = TPW // TCH


def _sc_hist_kernel(ei_hbm, typ_hbm, out_hbm,
                    e0, t0, e1, t1, hist_v, final_v,
                    sem0, sem1):
    wid = lax.axis_index("s")
    base = TPW * TILE * wid
    ebufs = (e0, e1)
    tbufs = (t0, t1)
    sems = (sem0, sem1)

    def start(c, parity):
        off = base + c * CW
        sem = sems[parity]
        return (pltpu.async_copy(ei_hbm.at[:, pl.ds(off, CW)],
                                 ebufs[parity], sem),
                pltpu.async_copy(typ_hbm.at[pl.ds(off, CW)],
                                 tbufs[parity], sem))

    start(0, 0)

    zeros = jnp.zeros((L,), jnp.float32)

    @plsc.parallel_loop(0, BINSP, unroll=8)
    def _(i):
        hist_v[pl.ds(i * L, L)] = zeros

    lane_off = lax.broadcasted_iota(jnp.int32, (L,), 0) * BINSP
    ones = jnp.ones((L,), jnp.float32)

    def drain(parity):
        pltpu.make_async_copy(ei_hbm.at[:, pl.ds(0, CW)], ebufs[parity],
                              sems[parity]).wait()
        pltpu.make_async_copy(typ_hbm.at[pl.ds(0, CW)], tbufs[parity],
                              sems[parity]).wait()

    def process(eb, tb, nvec):
        @plsc.parallel_loop(0, nvec, unroll=8)
        def _(i):
            o = i * L
            s = eb[0, pl.ds(o, L)]
            d = eb[1, pl.ds(o, L)]
            t = tb[pl.ds(o, L)]
            plsc.addupdate_scatter(
                hist_v, [(d * BP + t * BROWS) + (s + lane_off)], ones)

    def chunk_pair(j, c):
        drain(0)
        start(2 * j + 1, 1)
        process(e0, t0, CW // L)
        drain(1)

        @pl.when(2 * j + 2 < NCH)
        def _():
            start(2 * j + 2, 0)

        process(e1, t1, CW // L)
        return c

    lax.fori_loop(0, NCH // 2, chunk_pair, 0)

    @pl.when(wid < 2)
    def _():
        off = (NW * TPW + wid) * TILE
        cpe = pltpu.async_copy(ei_hbm.at[:, pl.ds(off, TILE)],
                               e0.at[:, pl.ds(0, TILE)], sem0)
        cpt = pltpu.async_copy(typ_hbm.at[pl.ds(off, TILE)],
                               t0.at[pl.ds(0, TILE)], sem0)
        cpe.wait()
        cpt.wait()
        process(e0, t0, TILE // L)

    for r in range(BROWS):
        @plsc.parallel_loop(0, BP // L, unroll=2)
        def _(i, r=r):
            o = r * BP + i * L
            acc = hist_v[pl.ds(o, L)]
            for l in range(1, L):
                acc = acc + hist_v[pl.ds(l * BINSP + o, L)]
            final_v[r, pl.ds(i * L, L)] = acc

    pltpu.sync_copy(final_v, out_hbm.at[wid])


def _make_sc_hist():
    return pl.kernel(
        _sc_hist_kernel,
        mesh=plsc.VectorSubcoreMesh(core_axis_name="c", subcore_axis_name="s",
                                    num_cores=1),
        out_type=jax.ShapeDtypeStruct((NW, BROWS, BP), jnp.float32),
        compiler_params=pltpu.CompilerParams(needs_layout_passes=False),
        scratch_types=[
            pltpu.VMEM((2, CW), jnp.int32),
            pltpu.VMEM((CW,), jnp.int32),
            pltpu.VMEM((2, CW), jnp.int32),
            pltpu.VMEM((CW,), jnp.int32),
            pltpu.VMEM((L * BINSP,), jnp.float32),
            pltpu.VMEM((BROWS, BP), jnp.float32),
            pltpu.SemaphoreType.DMA,
            pltpu.SemaphoreType.DMA,
        ],
    )


def _tc_finish_body(parts_ref, w1_ref, r1_ref, b1_ref, w2_ref, rt2_ref,
                    b2_ref, out_ref):
    counts = jnp.sum(parts_ref[...], axis=0)
    g = (lax.broadcasted_iota(jnp.int32, (BP, R), 0) // BROWS
         == lax.broadcasted_iota(jnp.int32, (BP, R), 1)).astype(jnp.float32)
    gt = (lax.broadcasted_iota(jnp.int32, (R, BP), 1) // BROWS
          == lax.broadcasted_iota(jnp.int32, (R, BP), 0)).astype(jnp.float32)
    cnt = jnp.dot(counts, g, preferred_element_type=jnp.float32)
    denom = jnp.maximum(
        jnp.dot(cnt, gt, preferred_element_type=jnp.float32), 1.0)
    q = (counts / denom)[0:N]
    w1 = jnp.concatenate(
        [w1_ref[...], jnp.zeros((R, 1, HID), jnp.float32)], axis=1
    ).reshape(BP, HID)
    agg1 = jnp.dot(q, w1, preferred_element_type=jnp.float32)
    h = jnp.maximum(agg1 + r1_ref[...] + b1_ref[...].reshape(1, HID), 0.0)
    hp = jnp.concatenate([h, jnp.zeros((1, HID), jnp.float32)], axis=0)
    w2h = jnp.concatenate(
        [jnp.dot(hp, w2_ref[r], preferred_element_type=jnp.float32)
         for r in range(R)], axis=0)
    acc = (jnp.dot(q, w2h, preferred_element_type=jnp.float32)
           + jnp.dot(h, rt2_ref[...], preferred_element_type=jnp.float32)
           + b2_ref[...].reshape(1, OUT))
    m = jnp.max(acc, axis=1, keepdims=True)
    e = jnp.exp(acc - m)
    lse = jnp.log(jnp.sum(e, axis=1, keepdims=True))
    out_ref[...] = acc - m - lse


def kernel(x, edge_index, edge_type, weight1, root1, bias1, weight2, root2,
           bias2):
    del x
    parts = _make_sc_hist()(edge_index, edge_type)
    return pl.pallas_call(
        _tc_finish_body,
        out_shape=jax.ShapeDtypeStruct((N, OUT), jnp.float32),
    )(parts, weight1, root1, bias1, weight2, root2, bias2)


# speedup vs baseline: 1.3581x; 1.1366x over previous
import functools

import jax
import jax.numpy as jnp
from jax import lax
from jax.experimental import pallas as pl
from jax.experimental.pallas import tpu as pltpu
from jax.experimental.pallas import tpu_sc as plsc

N = 7
R = 16
E = 640000
HID = 16
OUT = 8
RS = R * N
BP = 128
BROWS = 8
BINSP = BROWS * BP
L = 16
NW = 32
TILE = 512
NTILES = E // TILE
TPW = NTILES // NW
TCH = 13
CW = TCH * TILE
NCH = TPW // TCH


def _sc_hist_kernel(ei_hbm, typ_hbm, out_hbm,
                    e0, t0, e1, t1, hist_v, final_v,
                    sem0, sem1):
    wid = lax.axis_index("s") * 2 + lax.axis_index("c")
    base = TPW * TILE * wid
    ebufs = (e0, e1)
    tbufs = (t0, t1)
    sems = (sem0, sem1)

    def start(c, parity):
        off = base + c * CW
        sem = sems[parity]
        return (pltpu.async_copy(ei_hbm.at[:, pl.ds(off, CW)],
                                 ebufs[parity], sem),
                pltpu.async_copy(typ_hbm.at[pl.ds(off, CW)],
                                 tbufs[parity], sem))

    start(0, 0)

    zeros = jnp.zeros((L,), jnp.float32)

    @plsc.parallel_loop(0, BINSP, unroll=8)
    def _(i):
        hist_v[pl.ds(i * L, L)] = zeros

    lane_off = lax.broadcasted_iota(jnp.int32, (L,), 0) * BINSP
    ones = jnp.ones((L,), jnp.float32)

    def drain(parity):
        pltpu.make_async_copy(ei_hbm.at[:, pl.ds(0, CW)], ebufs[parity],
                              sems[parity]).wait()
        pltpu.make_async_copy(typ_hbm.at[pl.ds(0, CW)], tbufs[parity],
                              sems[parity]).wait()

    def process(eb, tb, nvec):
        @plsc.parallel_loop(0, nvec, unroll=8)
        def _(i):
            o = i * L
            s = eb[0, pl.ds(o, L)]
            d = eb[1, pl.ds(o, L)]
            t = tb[pl.ds(o, L)]
            plsc.addupdate_scatter(
                hist_v, [(d * BP + t * BROWS) + (s + lane_off)], ones)

    drain(0)
    start(1, 1)
    process(e0, t0, CW // L)
    drain(1)
    start(2, 0)
    process(e1, t1, CW // L)
    drain(0)
    process(e0, t0, CW // L)

    @pl.when(wid < 2)
    def _():
        off = (NW * TPW + wid) * TILE
        cpe = pltpu.async_copy(ei_hbm.at[:, pl.ds(off, TILE)],
                               e1.at[:, pl.ds(0, TILE)], sem1)
        cpt = pltpu.async_copy(typ_hbm.at[pl.ds(off, TILE)],
                               t1.at[pl.ds(0, TILE)], sem1)
        cpe.wait()
        cpt.wait()
        process(e1, t1, TILE // L)

    for r in range(BROWS):
        @plsc.parallel_loop(0, BP // L, unroll=2)
        def _(i, r=r):
            o = r * BP + i * L
            acc = hist_v[pl.ds(o, L)]
            for l in range(1, L):
                acc = acc + hist_v[pl.ds(l * BINSP + o, L)]
            final_v[r, pl.ds(i * L, L)] = acc

    pltpu.sync_copy(final_v, out_hbm.at[wid])


def _make_sc_hist():
    return pl.kernel(
        _sc_hist_kernel,
        mesh=plsc.VectorSubcoreMesh(core_axis_name="c", subcore_axis_name="s",
                                    num_cores=1),
        out_type=jax.ShapeDtypeStruct((NW, BROWS, BP), jnp.float32),
        compiler_params=pltpu.CompilerParams(needs_layout_passes=False),
        scratch_types=[
            pltpu.VMEM((2, CW), jnp.int32),
            pltpu.VMEM((CW,), jnp.int32),
            pltpu.VMEM((2, CW), jnp.int32),
            pltpu.VMEM((CW,), jnp.int32),
            pltpu.VMEM((L * BINSP,), jnp.float32),
            pltpu.VMEM((BROWS, BP), jnp.float32),
            pltpu.SemaphoreType.DMA,
            pltpu.SemaphoreType.DMA,
        ],
    )


def _tc_finish_body(parts_ref, w1_ref, r1_ref, b1_ref, w2_ref, rt2_ref,
                    b2_ref, out_ref):
    counts = jnp.sum(parts_ref[...], axis=0)
    g = (lax.broadcasted_iota(jnp.int32, (BP, R), 0) // BROWS
         == lax.broadcasted_iota(jnp.int32, (BP, R), 1)).astype(jnp.float32)
    gt = (lax.broadcasted_iota(jnp.int32, (R, BP), 1) // BROWS
          == lax.broadcasted_iota(jnp.int32, (R, BP), 0)).astype(jnp.float32)
    cnt = jnp.dot(counts, g, preferred_element_type=jnp.float32)
    denom = jnp.maximum(
        jnp.dot(cnt, gt, preferred_element_type=jnp.float32), 1.0)
    q = (counts / denom)[0:N]
    w1 = jnp.concatenate(
        [w1_ref[...], jnp.zeros((R, 1, HID), jnp.float32)], axis=1
    ).reshape(BP, HID)
    agg1 = jnp.dot(q, w1, preferred_element_type=jnp.float32)
    h = jnp.maximum(agg1 + r1_ref[...] + b1_ref[...].reshape(1, HID), 0.0)
    hp = jnp.concatenate([h, jnp.zeros((1, HID), jnp.float32)], axis=0)
    w2h = jnp.concatenate(
        [jnp.dot(hp, w2_ref[r], preferred_element_type=jnp.float32)
         for r in range(R)], axis=0)
    acc = (jnp.dot(q, w2h, preferred_element_type=jnp.float32)
           + jnp.dot(h, rt2_ref[...], preferred_element_type=jnp.float32)
           + b2_ref[...].reshape(1, OUT))
    m = jnp.max(acc, axis=1, keepdims=True)
    e = jnp.exp(acc - m)
    lse = jnp.log(jnp.sum(e, axis=1, keepdims=True))
    out_ref[...] = acc - m - lse


def kernel(x, edge_index, edge_type, weight1, root1, bias1, weight2, root2,
           bias2):
    del x
    parts = _make_sc_hist()(edge_index, edge_type)
    return pl.pallas_call(
        _tc_finish_body,
        out_shape=jax.ShapeDtypeStruct((N, OUT), jnp.float32),
    )(parts, weight1, root1, bias1, weight2, root2, bias2)


# speedup vs baseline: 1.3925x; 1.0253x over previous
import functools

import jax
import jax.numpy as jnp
from jax import lax
from jax.experimental import pallas as pl
from jax.experimental.pallas import tpu as pltpu
from jax.experimental.pallas import tpu_sc as plsc

N = 7
R = 16
E = 640000
HID = 16
OUT = 8
RS = R * N
BP = 128
BROWS = 8
BINSP = BROWS * BP
L = 16
NW = 32
TILE = 512
NTILES = E // TILE
TPW = NTILES // NW
TCH = 13
CW = TCH * TILE
NCH = TPW // TCH


def _sc_hist_kernel(ei_hbm, typ_hbm, out_hbm,
                    ebuf, tbuf, hist_v, final_v,
                    sem0, sem1):
    wid = lax.axis_index("s") * 2 + lax.axis_index("c")
    base = TPW * TILE * wid
    sems = (sem0, sem1)

    def start(c, parity):
        off = base + c * CW
        half = parity * CW
        sem = sems[parity]
        pltpu.async_copy(ei_hbm.at[:, pl.ds(off, CW)],
                         ebuf.at[:, pl.ds(half, CW)], sem)
        pltpu.async_copy(typ_hbm.at[pl.ds(off, CW)],
                         tbuf.at[pl.ds(half, CW)], sem)

    def drain(parity):
        pltpu.make_async_copy(ei_hbm.at[:, pl.ds(0, CW)],
                              ebuf.at[:, pl.ds(0, CW)], sems[parity]).wait()
        pltpu.make_async_copy(typ_hbm.at[pl.ds(0, CW)],
                              tbuf.at[pl.ds(0, CW)], sems[parity]).wait()

    start(0, 0)

    zeros = jnp.zeros((L,), jnp.float32)

    @plsc.parallel_loop(0, BINSP, unroll=8)
    def _(i):
        hist_v[pl.ds(i * L, L)] = zeros

    lane_off = lax.broadcasted_iota(jnp.int32, (L,), 0) * BINSP
    ones = jnp.ones((L,), jnp.float32)

    def process(half, nvec):
        @plsc.parallel_loop(0, nvec, unroll=8)
        def _(i):
            o = half + i * L
            s = ebuf[0, pl.ds(o, L)]
            d = ebuf[1, pl.ds(o, L)]
            t = tbuf[pl.ds(o, L)]
            plsc.addupdate_scatter(
                hist_v, [(d * BP + t * BROWS) + (s + lane_off)], ones)

    def chunk_body(c, carry):
        par = c & 1

        @pl.when(par == 0)
        def _():
            drain(0)

        @pl.when(par == 1)
        def _():
            drain(1)

        @pl.when((c + 1 < NCH) & (par == 0))
        def _():
            start(c + 1, 1)

        @pl.when((c + 1 < NCH) & (par == 1))
        def _():
            start(c + 1, 0)

        process(par * CW, CW // L)
        return carry

    lax.fori_loop(0, NCH, chunk_body, 0)

    @pl.when(wid < 2)
    def _():
        off = (NW * TPW + wid) * TILE
        cpe = pltpu.async_copy(ei_hbm.at[:, pl.ds(off, TILE)],
                               ebuf.at[:, pl.ds(0, TILE)], sem0)
        cpt = pltpu.async_copy(typ_hbm.at[pl.ds(off, TILE)],
                               tbuf.at[pl.ds(0, TILE)], sem0)
        cpe.wait()
        cpt.wait()

        @plsc.parallel_loop(0, TILE // L, unroll=2)
        def _(i):
            o = i * L
            s = ebuf[0, pl.ds(o, L)]
            d = ebuf[1, pl.ds(o, L)]
            t = tbuf[pl.ds(o, L)]
            plsc.addupdate_scatter(
                hist_v, [(d * BP + t * BROWS) + (s + lane_off)], ones)

    for r in range(BROWS):
        @plsc.parallel_loop(0, BP // L)
        def _(i, r=r):
            o = r * BP + i * L
            acc = hist_v[pl.ds(o, L)]
            for l in range(1, L):
                acc = acc + hist_v[pl.ds(l * BINSP + o, L)]
            final_v[r, pl.ds(i * L, L)] = acc

    pltpu.sync_copy(final_v, out_hbm.at[wid])


def _make_sc_hist():
    return pl.kernel(
        _sc_hist_kernel,
        mesh=plsc.VectorSubcoreMesh(core_axis_name="c", subcore_axis_name="s",
                                    num_cores=1),
        out_type=jax.ShapeDtypeStruct((NW, BROWS, BP), jnp.float32),
        compiler_params=pltpu.CompilerParams(needs_layout_passes=False),
        scratch_types=[
            pltpu.VMEM((2, 2 * CW), jnp.int32),
            pltpu.VMEM((2 * CW,), jnp.int32),
            pltpu.VMEM((L * BINSP,), jnp.float32),
            pltpu.VMEM((BROWS, BP), jnp.float32),
            pltpu.SemaphoreType.DMA,
            pltpu.SemaphoreType.DMA,
        ],
    )


def _tc_finish_body(parts_ref, w1_ref, r1_ref, b1_ref, w2_ref, rt2_ref,
                    b2_ref, out_ref):
    counts = jnp.sum(parts_ref[...], axis=0)
    g = (lax.broadcasted_iota(jnp.int32, (BP, R), 0) // BROWS
         == lax.broadcasted_iota(jnp.int32, (BP, R), 1)).astype(jnp.float32)
    gt = (lax.broadcasted_iota(jnp.int32, (R, BP), 1) // BROWS
          == lax.broadcasted_iota(jnp.int32, (R, BP), 0)).astype(jnp.float32)
    cnt = jnp.dot(counts, g, preferred_element_type=jnp.float32)
    denom = jnp.maximum(
        jnp.dot(cnt, gt, preferred_element_type=jnp.float32), 1.0)
    q = (counts / denom)[0:N]
    w1 = jnp.concatenate(
        [w1_ref[...], jnp.zeros((R, 1, HID), jnp.float32)], axis=1
    ).reshape(BP, HID)
    agg1 = jnp.dot(q, w1, preferred_element_type=jnp.float32)
    h = jnp.maximum(agg1 + r1_ref[...] + b1_ref[...].reshape(1, HID), 0.0)
    hp = jnp.concatenate([h, jnp.zeros((1, HID), jnp.float32)], axis=0)
    w2h = jnp.concatenate(
        [jnp.dot(hp, w2_ref[r], preferred_element_type=jnp.float32)
         for r in range(R)], axis=0)
    acc = (jnp.dot(q, w2h, preferred_element_type=jnp.float32)
           + jnp.dot(h, rt2_ref[...], preferred_element_type=jnp.float32)
           + b2_ref[...].reshape(1, OUT))
    m = jnp.max(acc, axis=1, keepdims=True)
    e = jnp.exp(acc - m)
    lse = jnp.log(jnp.sum(e, axis=1, keepdims=True))
    out_ref[...] = acc - m - lse


def kernel(x, edge_index, edge_type, weight1, root1, bias1, weight2, root2,
           bias2):
    del x
    parts = _make_sc_hist()(edge_index, edge_type)
    return pl.pallas_call(
        _tc_finish_body,
        out_shape=jax.ShapeDtypeStruct((N, OUT), jnp.float32),
    )(parts, weight1, root1, bias1, weight2, root2, bias2)
